# baseline (speedup 1.0000x reference)
import functools

import jax
import jax.numpy as jnp
from jax.experimental import pallas as pl
from jax.experimental.pallas import tpu as pltpu


def _fused_kernel(sw1_ref, sb1_ref, sw2_ref, sb2_ref, sw3_ref, sb3_ref,
                  wc1_ref, bc1_ref, wc2_ref, bc2_ref, wl_ref, bl_ref,
                  wc3_ref, bc3_ref, wc4_ref, bc4_ref,
                  x_ref, h_ref, c_ref,
                  y_ref, ho_ref, co_ref,
                  pads0_ref, pads1_ref,
                  *, C, W, P, PADL, BB):
    bf16 = jnp.bfloat16
    f32 = jnp.float32
    L0 = PADL
    L1 = PADL + W

    col_id = jax.lax.broadcasted_iota(jnp.int32, (1, P), 1) % W
    m_left = (col_id != 0).astype(bf16)
    m_right = (col_id != W - 1).astype(bf16)

    pads0_ref[:, :, 0:L0] = jnp.zeros((BB, 6 * C, L0), bf16)
    pads0_ref[:, :, L0 + P:] = jnp.zeros((BB, 6 * C, PADL), bf16)
    pads1_ref[:, :, 0:L1] = jnp.zeros((BB, 6 * C, L1), bf16)
    pads1_ref[:, :, L1 + P:] = jnp.zeros((BB, 6 * C, PADL - W), bf16)

    def conv(b, a, w_ref, b_ref):
        cc = a.shape[0]
        a_l = m_left * jnp.roll(a, 1, axis=1)
        a_r = m_right * jnp.roll(a, -1, axis=1)
        pads0_ref[b, cc:2 * cc, L0:L0 + P] = a
        pads0_ref[b, 0:cc, L0:L0 + P] = a_l
        pads0_ref[b, 2 * cc:3 * cc, L0:L0 + P] = a_r
        pads1_ref[b, 0:cc, L1:L1 + P] = a_l
        pads1_ref[b, cc:2 * cc, L1:L1 + P] = a
        pads1_ref[b, 2 * cc:3 * cc, L1:L1 + P] = a_r
        acc = b_ref[...]
        acc = acc + jnp.dot(w_ref[:, 0:3 * cc],
                            pads1_ref[b, 0:3 * cc, L1 - W:L1 - W + P],
                            preferred_element_type=f32)
        acc = acc + jnp.dot(w_ref[:, 3 * cc:6 * cc],
                            pads0_ref[b, 0:3 * cc, L0:L0 + P],
                            preferred_element_type=f32)
        acc = acc + jnp.dot(w_ref[:, 6 * cc:9 * cc],
                            pads1_ref[b, 0:3 * cc, L1 + W:L1 + W + P],
                            preferred_element_type=f32)
        return acc

    def stage_se(b):
        x32 = x_ref[b]
        se_in = jnp.mean(x32, axis=1, keepdims=True)
        z = jnp.maximum(jnp.dot(sw1_ref[...], se_in,
                                preferred_element_type=f32) + sb1_ref[...], 0.0)
        z = jnp.maximum(jnp.dot(sw2_ref[...], z,
                                preferred_element_type=f32) + sb2_ref[...], 0.0)
        s = jax.nn.sigmoid(jnp.dot(sw3_ref[...], z,
                                   preferred_element_type=f32) + sb3_ref[...])
        return (x32 * s).astype(bf16)

    def stage_convs12(b, a0):
        a1 = jnp.maximum(conv(b, a0, wc1_ref, bc1_ref), 0.0).astype(bf16)
        return jnp.maximum(conv(b, a1, wc2_ref, bc2_ref), 0.0).astype(bf16)

    def stage_lstm(b, a2):
        xh = jnp.concatenate([a2, h_ref[b]], axis=0)
        gates = conv(b, xh, wl_ref, bl_ref)
        sig = lambda v: 0.5 * jnp.tanh(0.5 * v) + 0.5
        gi = sig(gates[0 * C:1 * C])
        gf = sig(gates[1 * C:2 * C])
        gg = jnp.tanh(gates[2 * C:3 * C])
        go = sig(gates[3 * C:4 * C])
        c_new = gf * c_ref[b] + gi * gg
        h_new = go * jnp.tanh(c_new)
        co_ref[b] = c_new
        ho_ref[b] = h_new
        return h_new

    def stage_convs34(b, h_new):
        a3 = jnp.maximum(conv(b, h_new.astype(bf16), wc3_ref, bc3_ref),
                         0.0).astype(bf16)
        y_ref[b] = jnp.maximum(conv(b, a3, wc4_ref, bc4_ref), 0.0)

    a0s = [stage_se(b) for b in range(BB)]
    a2s = [stage_convs12(b, a0s[b]) for b in range(BB)]
    hns = [stage_lstm(b, a2s[b]) for b in range(BB)]
    for b in range(BB):
        stage_convs34(b, hns[b])


def kernel(se_w1, se_b1, se_w2, se_b2, se_w3, se_b3,
           conv1_w, conv1_b, conv2_w, conv2_b,
           lstm_w, lstm_b, conv3_w, conv3_b, conv4_w, conv4_b,
           x, h, c):
    B, C, H, W = x.shape
    P = H * W
    PADL = 2 * W
    bf16 = jnp.bfloat16
    f32 = jnp.float32

    def tconv(w):
        co = w.shape[3]
        return jnp.transpose(w, (3, 0, 1, 2)).reshape(co, -1).astype(bf16)

    wc1, wc2, wc3, wc4 = map(tconv, (conv1_w, conv2_w, conv3_w, conv4_w))
    wl = tconv(lstm_w)
    tb = lambda b: jnp.transpose(b)
    bc1, bc2, bc3, bc4, bl = map(tb, (conv1_b, conv2_b, conv3_b, conv4_b,
                                      lstm_b))
    sw1, sw2, sw3 = (jnp.transpose(w) for w in (se_w1, se_w2, se_w3))
    sb1, sb2, sb3 = map(tb, (se_b1, se_b2, se_b3))

    xr, cr = x.reshape(B, C, P), c.reshape(B, C, P)
    hr = h.reshape(B, C, P).astype(bf16)

    def full_spec(arr):
        nd = arr.ndim
        return pl.BlockSpec(arr.shape, lambda i, _nd=nd: (0,) * _nd)

    BB = 2 if B % 2 == 0 else 1
    plane_spec = pl.BlockSpec((BB, C, P), lambda i: (i, 0, 0))

    weights = (sw1, sb1, sw2, sb2, sw3, sb3,
               wc1, bc1, wc2, bc2, wl, bl, wc3, bc3, wc4, bc4)

    flops = 2 * B * P * 9 * C * C * 12
    trans = 7 * B * P * C
    bytes_accessed = 4 * 6 * B * C * P

    out = pl.pallas_call(
        functools.partial(_fused_kernel, C=C, W=W, P=P, PADL=PADL, BB=BB),
        out_shape=(jax.ShapeDtypeStruct((B, C, P), f32),
                   jax.ShapeDtypeStruct((B, C, P), f32),
                   jax.ShapeDtypeStruct((B, C, P), f32)),
        grid_spec=pltpu.PrefetchScalarGridSpec(
            num_scalar_prefetch=0,
            grid=(B // BB,),
            in_specs=[full_spec(w) for w in weights]
                     + [plane_spec, plane_spec, plane_spec],
            out_specs=[plane_spec, plane_spec, plane_spec],
            scratch_shapes=[
                pltpu.VMEM((BB, 6 * C, P + 2 * PADL), bf16),
                pltpu.VMEM((BB, 6 * C, P + 2 * PADL), bf16),
            ],
        ),
        compiler_params=pltpu.CompilerParams(
            dimension_semantics=("parallel",)),
        cost_estimate=pl.CostEstimate(flops=flops, transcendentals=trans,
                                      bytes_accessed=bytes_accessed),
    )(*weights, xr, hr, cr)

    y, h_new, c_new = out
    shape = (B, C, H, W)
    return y.reshape(shape), h_new.reshape(shape), c_new.reshape(shape)


# speedup vs baseline: 1.0206x; 1.0206x over previous
import functools

import jax
import jax.numpy as jnp
from jax.experimental import pallas as pl
from jax.experimental.pallas import tpu as pltpu


def _fused_kernel(sw1_ref, sb1_ref, sw2_ref, sb2_ref, sw3_ref, sb3_ref,
                  wc1_ref, bc1_ref, wc2_ref, bc2_ref, wl_ref, bl_ref,
                  wc3_ref, bc3_ref, wc4_ref, bc4_ref,
                  x_ref, h_ref, c_ref,
                  y_ref, ho_ref, co_ref,
                  pads0_ref, pads1_ref,
                  *, C, W, P, PADL, BB):
    bf16 = jnp.bfloat16
    f32 = jnp.float32
    L0 = PADL
    L1 = PADL + W

    col_id = jax.lax.broadcasted_iota(jnp.int32, (1, P), 1) % W
    m_left = (col_id != 0).astype(bf16)
    m_right = (col_id != W - 1).astype(bf16)

    pads0_ref[:, :, 0:L0] = jnp.zeros((BB, 6 * C, L0), bf16)
    pads0_ref[:, :, L0 + P:] = jnp.zeros((BB, 6 * C, PADL), bf16)
    pads1_ref[:, :, 0:L1] = jnp.zeros((BB, 6 * C, L1), bf16)
    pads1_ref[:, :, L1 + P:] = jnp.zeros((BB, 6 * C, PADL - W), bf16)

    def conv(b, a, w_ref, b_ref):
        cc = a.shape[0]
        a_l = m_left * jnp.roll(a, 1, axis=1)
        a_r = m_right * jnp.roll(a, -1, axis=1)
        pads0_ref[b, cc:2 * cc, L0:L0 + P] = a
        pads0_ref[b, 0:cc, L0:L0 + P] = a_l
        pads0_ref[b, 2 * cc:3 * cc, L0:L0 + P] = a_r
        pads1_ref[b, 0:cc, L1:L1 + P] = a_l
        pads1_ref[b, cc:2 * cc, L1:L1 + P] = a
        pads1_ref[b, 2 * cc:3 * cc, L1:L1 + P] = a_r
        dg = (((0,), (0,)), ((), ()))
        acc = b_ref[...]
        acc = acc + jax.lax.dot_general(
            w_ref[0 * 3 * cc:1 * 3 * cc, :],
            pads1_ref[b, 0:3 * cc, L1 - W:L1 - W + P],
            dg, preferred_element_type=f32)
        acc = acc + jax.lax.dot_general(
            w_ref[1 * 3 * cc:2 * 3 * cc, :],
            pads0_ref[b, 0:3 * cc, L0:L0 + P],
            dg, preferred_element_type=f32)
        acc = acc + jax.lax.dot_general(
            w_ref[2 * 3 * cc:3 * 3 * cc, :],
            pads1_ref[b, 0:3 * cc, L1 + W:L1 + W + P],
            dg, preferred_element_type=f32)
        return acc

    def stage_se(b):
        dg = (((0,), (0,)), ((), ()))
        x32 = x_ref[b]
        se_in = jnp.mean(x32, axis=1, keepdims=True)
        z = jnp.maximum(jax.lax.dot_general(sw1_ref[...], se_in, dg,
                        preferred_element_type=f32) + sb1_ref[...], 0.0)
        z = jnp.maximum(jax.lax.dot_general(sw2_ref[...], z, dg,
                        preferred_element_type=f32) + sb2_ref[...], 0.0)
        s = jax.nn.sigmoid(jax.lax.dot_general(sw3_ref[...], z, dg,
                           preferred_element_type=f32) + sb3_ref[...])
        return (x32 * s).astype(bf16)

    def stage_convs12(b, a0):
        a1 = jnp.maximum(conv(b, a0, wc1_ref, bc1_ref), 0.0).astype(bf16)
        return jnp.maximum(conv(b, a1, wc2_ref, bc2_ref), 0.0).astype(bf16)

    def stage_lstm(b, a2):
        xh = jnp.concatenate([a2, h_ref[b].astype(bf16)], axis=0)
        gates = conv(b, xh, wl_ref, bl_ref)
        sig = lambda v: 0.5 * jnp.tanh(0.5 * v) + 0.5
        gi = sig(gates[0 * C:1 * C])
        gf = sig(gates[1 * C:2 * C])
        gg = jnp.tanh(gates[2 * C:3 * C])
        go = sig(gates[3 * C:4 * C])
        c_new = gf * c_ref[b] + gi * gg
        h_new = go * jnp.tanh(c_new)
        co_ref[b] = c_new
        ho_ref[b] = h_new
        return h_new

    def stage_convs34(b, h_new):
        a3 = jnp.maximum(conv(b, h_new.astype(bf16), wc3_ref, bc3_ref),
                         0.0).astype(bf16)
        y_ref[b] = jnp.maximum(conv(b, a3, wc4_ref, bc4_ref), 0.0)

    a0s = [stage_se(b) for b in range(BB)]
    a2s = [stage_convs12(b, a0s[b]) for b in range(BB)]
    hns = [stage_lstm(b, a2s[b]) for b in range(BB)]
    for b in range(BB):
        stage_convs34(b, hns[b])


def kernel(se_w1, se_b1, se_w2, se_b2, se_w3, se_b3,
           conv1_w, conv1_b, conv2_w, conv2_b,
           lstm_w, lstm_b, conv3_w, conv3_b, conv4_w, conv4_b,
           x, h, c):
    B, C, H, W = x.shape
    P = H * W
    PADL = 2 * W
    bf16 = jnp.bfloat16
    f32 = jnp.float32

    def tconv(w):
        return w.reshape(-1, w.shape[3]).astype(bf16)

    wc1, wc2, wc3, wc4 = map(tconv, (conv1_w, conv2_w, conv3_w, conv4_w))
    wl = tconv(lstm_w)
    tb = lambda b: b.reshape(-1, 1)
    bc1, bc2, bc3, bc4, bl = map(tb, (conv1_b, conv2_b, conv3_b, conv4_b,
                                      lstm_b))
    sw1, sw2, sw3 = se_w1, se_w2, se_w3
    sb1, sb2, sb3 = map(tb, (se_b1, se_b2, se_b3))

    xr, hr, cr = (t.reshape(B, C, P) for t in (x, h, c))

    def full_spec(arr):
        nd = arr.ndim
        return pl.BlockSpec(arr.shape, lambda i, _nd=nd: (0,) * _nd)

    BB = 2 if B % 2 == 0 else 1
    plane_spec = pl.BlockSpec((BB, C, P), lambda i: (i, 0, 0))

    weights = (sw1, sb1, sw2, sb2, sw3, sb3,
               wc1, bc1, wc2, bc2, wl, bl, wc3, bc3, wc4, bc4)

    flops = 2 * B * P * 9 * C * C * 12
    trans = 7 * B * P * C
    bytes_accessed = 4 * 6 * B * C * P

    out = pl.pallas_call(
        functools.partial(_fused_kernel, C=C, W=W, P=P, PADL=PADL, BB=BB),
        out_shape=(jax.ShapeDtypeStruct((B, C, P), f32),
                   jax.ShapeDtypeStruct((B, C, P), f32),
                   jax.ShapeDtypeStruct((B, C, P), f32)),
        grid_spec=pltpu.PrefetchScalarGridSpec(
            num_scalar_prefetch=0,
            grid=(B // BB,),
            in_specs=[full_spec(w) for w in weights]
                     + [plane_spec, plane_spec, plane_spec],
            out_specs=[plane_spec, plane_spec, plane_spec],
            scratch_shapes=[
                pltpu.VMEM((BB, 6 * C, P + 2 * PADL), bf16),
                pltpu.VMEM((BB, 6 * C, P + 2 * PADL), bf16),
            ],
        ),
        compiler_params=pltpu.CompilerParams(
            dimension_semantics=("parallel",)),
        cost_estimate=pl.CostEstimate(flops=flops, transcendentals=trans,
                                      bytes_accessed=bytes_accessed),
    )(*weights, xr, hr, cr)

    y, h_new, c_new = out
    shape = (B, C, H, W)
    return y.reshape(shape), h_new.reshape(shape), c_new.reshape(shape)


# speedup vs baseline: 1.0333x; 1.0124x over previous
import functools

import jax
import jax.numpy as jnp
from jax.experimental import pallas as pl
from jax.experimental.pallas import tpu as pltpu


def _fused_kernel(sw1_ref, sb1_ref, sw2_ref, sb2_ref, sw3_ref, sb3_ref,
                  wc1_ref, bc1_ref, wc2_ref, bc2_ref, wl_ref, bl_ref,
                  wc3_ref, bc3_ref, wc4_ref, bc4_ref,
                  x_ref, h_ref, c_ref,
                  y_ref, ho_ref, co_ref,
                  pads0_ref, pads1_ref,
                  *, C, W, P, PADL, BB):
    bf16 = jnp.bfloat16
    f32 = jnp.float32
    L0 = PADL
    L1 = PADL + W

    col_id = jax.lax.broadcasted_iota(jnp.int32, (1, P), 1) % W
    m_left = (col_id != 0).astype(bf16)
    m_right = (col_id != W - 1).astype(bf16)

    pads0_ref[:, :, 0:L0] = jnp.zeros((BB, 6 * C, L0), bf16)
    pads0_ref[:, :, L0 + P:] = jnp.zeros((BB, 6 * C, PADL), bf16)
    pads1_ref[:, :, 0:L1] = jnp.zeros((BB, 6 * C, L1), bf16)
    pads1_ref[:, :, L1 + P:] = jnp.zeros((BB, 6 * C, PADL - W), bf16)

    def conv(b, a, w_ref, b_ref):
        cc = a.shape[0]
        a_l = m_left * jnp.roll(a, 1, axis=1)
        a_r = m_right * jnp.roll(a, -1, axis=1)
        pads0_ref[b, cc:2 * cc, L0:L0 + P] = a
        pads0_ref[b, 0:cc, L0:L0 + P] = a_l
        pads0_ref[b, 2 * cc:3 * cc, L0:L0 + P] = a_r
        pads1_ref[b, 0:cc, L1:L1 + P] = a_l
        pads1_ref[b, cc:2 * cc, L1:L1 + P] = a
        pads1_ref[b, 2 * cc:3 * cc, L1:L1 + P] = a_r
        acc = b_ref[...]
        acc = acc + jnp.dot(w_ref[:, 0:3 * cc],
                            pads1_ref[b, 0:3 * cc, L1 - W:L1 - W + P],
                            preferred_element_type=f32)
        acc = acc + jnp.dot(w_ref[:, 3 * cc:6 * cc],
                            pads0_ref[b, 0:3 * cc, L0:L0 + P],
                            preferred_element_type=f32)
        acc = acc + jnp.dot(w_ref[:, 6 * cc:9 * cc],
                            pads1_ref[b, 0:3 * cc, L1 + W:L1 + W + P],
                            preferred_element_type=f32)
        return acc

    def stage_se(b):
        x32 = x_ref[b]
        se_in = jnp.mean(x32, axis=1, keepdims=True)
        z = jnp.maximum(jnp.dot(sw1_ref[...], se_in,
                                preferred_element_type=f32) + sb1_ref[...], 0.0)
        z = jnp.maximum(jnp.dot(sw2_ref[...], z,
                                preferred_element_type=f32) + sb2_ref[...], 0.0)
        s = jax.nn.sigmoid(jnp.dot(sw3_ref[...], z,
                                   preferred_element_type=f32) + sb3_ref[...])
        return (x32 * s).astype(bf16)

    def stage_convs12(b, a0):
        a1 = jnp.maximum(conv(b, a0, wc1_ref, bc1_ref), 0.0).astype(bf16)
        return jnp.maximum(conv(b, a1, wc2_ref, bc2_ref), 0.0).astype(bf16)

    def stage_lstm(b, a2):
        xh = jnp.concatenate([a2, h_ref[b].astype(bf16)], axis=0)
        gates = conv(b, xh, wl_ref, bl_ref)
        sig = lambda v: 0.5 * jnp.tanh(0.5 * v) + 0.5
        gi = sig(gates[0 * C:1 * C])
        gf = sig(gates[1 * C:2 * C])
        gg = jnp.tanh(gates[2 * C:3 * C])
        go = sig(gates[3 * C:4 * C])
        c_new = gf * c_ref[b] + gi * gg
        h_new = go * jnp.tanh(c_new)
        co_ref[b] = c_new
        ho_ref[b] = h_new
        return h_new

    def stage_convs34(b, h_new):
        a3 = jnp.maximum(conv(b, h_new.astype(bf16), wc3_ref, bc3_ref),
                         0.0).astype(bf16)
        y_ref[b] = jnp.maximum(conv(b, a3, wc4_ref, bc4_ref), 0.0)

    a0s = [stage_se(b) for b in range(BB)]
    a2s = [stage_convs12(b, a0s[b]) for b in range(BB)]
    hns = [stage_lstm(b, a2s[b]) for b in range(BB)]
    for b in range(BB):
        stage_convs34(b, hns[b])


def kernel(se_w1, se_b1, se_w2, se_b2, se_w3, se_b3,
           conv1_w, conv1_b, conv2_w, conv2_b,
           lstm_w, lstm_b, conv3_w, conv3_b, conv4_w, conv4_b,
           x, h, c):
    B, C, H, W = x.shape
    P = H * W
    PADL = 2 * W
    bf16 = jnp.bfloat16
    f32 = jnp.float32

    def tconv(w):
        co = w.shape[3]
        return jnp.transpose(w, (3, 0, 1, 2)).reshape(co, -1).astype(bf16)

    wc1, wc2, wc3, wc4 = map(tconv, (conv1_w, conv2_w, conv3_w, conv4_w))
    wl = tconv(lstm_w)
    tb = lambda b: jnp.transpose(b)
    bc1, bc2, bc3, bc4, bl = map(tb, (conv1_b, conv2_b, conv3_b, conv4_b,
                                      lstm_b))
    sw1, sw2, sw3 = (jnp.transpose(w) for w in (se_w1, se_w2, se_w3))
    sb1, sb2, sb3 = map(tb, (se_b1, se_b2, se_b3))

    xr, hr, cr = (t.reshape(B, C, P) for t in (x, h, c))

    def full_spec(arr):
        nd = arr.ndim
        return pl.BlockSpec(arr.shape, lambda i, _nd=nd: (0,) * _nd)

    BB = 2 if B % 2 == 0 else 1
    plane_spec = pl.BlockSpec((BB, C, P), lambda i: (i, 0, 0))

    weights = (sw1, sb1, sw2, sb2, sw3, sb3,
               wc1, bc1, wc2, bc2, wl, bl, wc3, bc3, wc4, bc4)

    flops = 2 * B * P * 9 * C * C * 12
    trans = 7 * B * P * C
    bytes_accessed = 4 * 6 * B * C * P

    out = pl.pallas_call(
        functools.partial(_fused_kernel, C=C, W=W, P=P, PADL=PADL, BB=BB),
        out_shape=(jax.ShapeDtypeStruct((B, C, P), f32),
                   jax.ShapeDtypeStruct((B, C, P), f32),
                   jax.ShapeDtypeStruct((B, C, P), f32)),
        grid_spec=pltpu.PrefetchScalarGridSpec(
            num_scalar_prefetch=0,
            grid=(B // BB,),
            in_specs=[full_spec(w) for w in weights]
                     + [plane_spec, plane_spec, plane_spec],
            out_specs=[plane_spec, plane_spec, plane_spec],
            scratch_shapes=[
                pltpu.VMEM((BB, 6 * C, P + 2 * PADL), bf16),
                pltpu.VMEM((BB, 6 * C, P + 2 * PADL), bf16),
            ],
        ),
        compiler_params=pltpu.CompilerParams(
            dimension_semantics=("parallel",)),
        cost_estimate=pl.CostEstimate(flops=flops, transcendentals=trans,
                                      bytes_accessed=bytes_accessed),
    )(*weights, xr, hr, cr)

    y, h_new, c_new = out
    shape = (B, C, H, W)
    return y.reshape(shape), h_new.reshape(shape), c_new.reshape(shape)
